# baseline (speedup 1.0000x reference)
import functools
import math

import jax
import jax.numpy as jnp
from jax import lax
from jax.experimental import pallas as pl
from jax.experimental.pallas import tpu as pltpu
from jax.experimental.pallas import tpu_sc as plsc

D_MODEL = 128
MAX_SEQ_LEN = 200
BATCH = 1024
_SCALE = math.sqrt(float(D_MODEL))

NC = 2
NS = 16
NW = NC * NS
ROWS_PER_W = BATCH // NW
GCHUNK = 2
GLEN = MAX_SEQ_LEN // GCHUNK
LANES = 16
NVEC = D_MODEL // LANES
NBUF = 4
AHEAD = 2
IBLK = 4
NIBLK = ROWS_PER_W // IBLK


def _pos_encoding():
    even_i = jnp.arange(0, D_MODEL, 2, dtype=jnp.float32)
    denominator = jnp.power(even_i, even_i / D_MODEL)
    position = jnp.arange(MAX_SEQ_LEN, dtype=jnp.float32).reshape(MAX_SEQ_LEN, 1)
    even_pe = jnp.sin(position / denominator)
    odd_pe = jnp.cos(position / denominator)
    stacked = jnp.stack([even_pe, odd_pe], axis=-1)
    return stacked.reshape(MAX_SEQ_LEN, D_MODEL)


def _embed_encode(x5, pe, table):
    mesh = plsc.VectorSubcoreMesh(core_axis_name="c", subcore_axis_name="s")

    scratch = [
        pltpu.VMEM((MAX_SEQ_LEN, D_MODEL), jnp.float32),
        pltpu.VMEM((2, IBLK, GCHUNK, GLEN), jnp.int32),
    ]
    scratch += [pltpu.VMEM((MAX_SEQ_LEN, D_MODEL), jnp.float32)
                for _ in range(NBUF)]
    scratch += [pltpu.SemaphoreType.DMA for _ in range(2 * NBUF + 1)]

    @functools.partial(
        pl.kernel,
        out_type=jax.ShapeDtypeStruct((BATCH, MAX_SEQ_LEN, D_MODEL), jnp.float32),
        mesh=mesh,
        scratch_types=scratch,
    )
    def k(x_hbm, pe_hbm, table_hbm, out_hbm, pe_v, idx_db, *rest):
        rows = rest[:NBUF]
        gsems = rest[NBUF:2 * NBUF]
        osems = rest[2 * NBUF:3 * NBUF]
        isem = rest[3 * NBUF]

        wid = lax.axis_index("s") * NC + lax.axis_index("c")
        base = wid * ROWS_PER_W
        pltpu.sync_copy(x_hbm.at[wid, 0], idx_db.at[0])
        pe_cp = pltpu.async_copy(pe_hbm, pe_v, isem)

        def prefetch_iblk(b):
            pltpu.async_copy(x_hbm.at[wid, b], idx_db.at[b % 2], isem)

        def wait_iblk(b):
            pltpu.make_async_copy(
                x_hbm.at[wid, b], idx_db.at[b % 2], isem).wait()

        def issue_gather(i, p):
            for g in range(GCHUNK):
                pltpu.async_copy(
                    table_hbm.at[idx_db.at[(i // IBLK) % 2, i % IBLK, g]],
                    rows[p].at[pl.ds(g * GLEN, GLEN)], gsems[p])

        def wait_gather(i, p):
            for g in range(GCHUNK):
                pltpu.make_async_copy(
                    table_hbm.at[idx_db.at[(i // IBLK) % 2, i % IBLK, g]],
                    rows[p].at[pl.ds(g * GLEN, GLEN)], gsems[p]).wait()

        def issue_out(i, p):
            pltpu.async_copy(rows[p], out_hbm.at[base + i], osems[p])

        def wait_out(i, p):
            pltpu.make_async_copy(rows[p], out_hbm.at[base + i], osems[p]).wait()

        SPLIT = 96

        def compute_pair_part(pa, pb, lo, hi):
            ra, rb = rows[pa], rows[pb]

            @plsc.parallel_loop(lo, hi, step=1)
            def _(r):
                for c in range(NVEC):
                    sl = pl.ds(c * LANES, LANES)
                    pev = pe_v[r, sl]
                    ra[r, sl] = ra[r, sl] * _SCALE + pev
                    rb[r, sl] = rb[r, sl] * _SCALE + pev

        def issue_out_part(i, p, lo, sz):
            pltpu.async_copy(rows[p].at[pl.ds(lo, sz)],
                             out_hbm.at[base + i, pl.ds(lo, sz)], osems[p])

        for i in range(AHEAD):
            issue_gather(i, i % NBUF)
        pe_cp.wait()
        for i in range(ROWS_PER_W):
            p = i % NBUF
            nxt = i + AHEAD
            if nxt < ROWS_PER_W:
                pn = nxt % NBUF
                if nxt % IBLK == 0 and nxt >= IBLK:
                    wait_iblk(nxt // IBLK)
                if nxt >= NBUF:
                    wait_out(nxt - NBUF, pn)
                issue_gather(nxt, pn)
            wait_gather(i, p)
            if i % IBLK == 0 and i // IBLK + 1 < NIBLK:
                prefetch_iblk(i // IBLK + 1)
            if i % 2 == 1:
                pp = (i - 1) % NBUF
                compute_pair_part(pp, p, 0, SPLIT)
                issue_out_part(i - 1, pp, 0, SPLIT)
                issue_out_part(i, p, 0, SPLIT)
                compute_pair_part(pp, p, SPLIT, MAX_SEQ_LEN)
                issue_out_part(i - 1, pp, SPLIT, MAX_SEQ_LEN - SPLIT)
                issue_out_part(i, p, SPLIT, MAX_SEQ_LEN - SPLIT)
        for i in range(ROWS_PER_W - NBUF, ROWS_PER_W):
            wait_out(i, i % NBUF)

    return k(x5, pe, table)


def kernel(x, table):
    x5 = x.reshape(NW, NIBLK, IBLK, GCHUNK, GLEN)
    pe = _pos_encoding()
    return _embed_encode(x5, pe, table)


# speedup vs baseline: 1.0045x; 1.0045x over previous
import functools
import math

import jax
import jax.numpy as jnp
from jax import lax
from jax.experimental import pallas as pl
from jax.experimental.pallas import tpu as pltpu
from jax.experimental.pallas import tpu_sc as plsc

D_MODEL = 128
MAX_SEQ_LEN = 200
BATCH = 1024
_SCALE = math.sqrt(float(D_MODEL))

NC = 2
NS = 16
NW = NC * NS
ROWS_PER_W = BATCH // NW
GCHUNK = 2
GLEN = MAX_SEQ_LEN // GCHUNK
LANES = 16
NVEC = D_MODEL // LANES
NBUF = 4
AHEAD = 2
IBLK = 4
NIBLK = ROWS_PER_W // IBLK


def _pos_encoding():
    even_i = jnp.arange(0, D_MODEL, 2, dtype=jnp.float32)
    denominator = jnp.power(even_i, even_i / D_MODEL)
    position = jnp.arange(MAX_SEQ_LEN, dtype=jnp.float32).reshape(MAX_SEQ_LEN, 1)
    even_pe = jnp.sin(position / denominator)
    odd_pe = jnp.cos(position / denominator)
    stacked = jnp.stack([even_pe, odd_pe], axis=-1)
    return stacked.reshape(MAX_SEQ_LEN, D_MODEL)


def _embed_encode(x5, pe, table):
    mesh = plsc.VectorSubcoreMesh(core_axis_name="c", subcore_axis_name="s")

    scratch = [
        pltpu.VMEM((MAX_SEQ_LEN, D_MODEL), jnp.float32),
        pltpu.VMEM((2, IBLK, GCHUNK, GLEN), jnp.int32),
    ]
    scratch += [pltpu.VMEM((MAX_SEQ_LEN, D_MODEL), jnp.float32)
                for _ in range(NBUF)]
    scratch += [pltpu.SemaphoreType.DMA for _ in range(2 * NBUF + 1)]

    @functools.partial(
        pl.kernel,
        out_type=jax.ShapeDtypeStruct((BATCH, MAX_SEQ_LEN, D_MODEL), jnp.float32),
        mesh=mesh,
        scratch_types=scratch,
    )
    def k(x_hbm, pe_hbm, table_hbm, out_hbm, pe_v, idx_db, *rest):
        rows = rest[:NBUF]
        gsems = rest[NBUF:2 * NBUF]
        osems = rest[2 * NBUF:3 * NBUF]
        isem = rest[3 * NBUF]

        wid = lax.axis_index("s") * NC + lax.axis_index("c")
        base = wid * ROWS_PER_W
        pltpu.sync_copy(x_hbm.at[wid, 0], idx_db.at[0])
        pe_cp = pltpu.async_copy(pe_hbm, pe_v, isem)

        def prefetch_iblk(b):
            pltpu.async_copy(x_hbm.at[wid, b], idx_db.at[b % 2], isem)

        def wait_iblk(b):
            pltpu.make_async_copy(
                x_hbm.at[wid, b], idx_db.at[b % 2], isem).wait()

        def issue_gather(i, p):
            for g in range(GCHUNK):
                pltpu.async_copy(
                    table_hbm.at[idx_db.at[(i // IBLK) % 2, i % IBLK, g]],
                    rows[p].at[pl.ds(g * GLEN, GLEN)], gsems[p])

        def wait_gather(i, p):
            for g in range(GCHUNK):
                pltpu.make_async_copy(
                    table_hbm.at[idx_db.at[(i // IBLK) % 2, i % IBLK, g]],
                    rows[p].at[pl.ds(g * GLEN, GLEN)], gsems[p]).wait()

        def issue_out(i, p):
            pltpu.async_copy(rows[p], out_hbm.at[base + i], osems[p])

        def wait_out(i, p):
            pltpu.make_async_copy(rows[p], out_hbm.at[base + i], osems[p]).wait()

        def compute_pair(pa, pb):
            ra, rb = rows[pa], rows[pb]

            @plsc.parallel_loop(0, MAX_SEQ_LEN, step=1)
            def _(r):
                for c in range(NVEC):
                    sl = pl.ds(c * LANES, LANES)
                    pev = pe_v[r, sl]
                    ra[r, sl] = ra[r, sl] * _SCALE + pev
                    rb[r, sl] = rb[r, sl] * _SCALE + pev

        for i in range(AHEAD):
            issue_gather(i, i % NBUF)
        pe_cp.wait()
        for i in range(ROWS_PER_W):
            p = i % NBUF
            nxt = i + AHEAD
            if nxt < ROWS_PER_W:
                pn = nxt % NBUF
                if nxt % IBLK == 0 and nxt >= IBLK:
                    wait_iblk(nxt // IBLK)
                if nxt >= NBUF:
                    wait_out(nxt - NBUF, pn)
                issue_gather(nxt, pn)
            wait_gather(i, p)
            if i % IBLK == 0 and i // IBLK + 1 < NIBLK:
                prefetch_iblk(i // IBLK + 1)
            if i % 2 == 1:
                compute_pair((i - 1) % NBUF, p)
                issue_out(i - 1, (i - 1) % NBUF)
                issue_out(i, p)
        for i in range(ROWS_PER_W - NBUF, ROWS_PER_W):
            wait_out(i, i % NBUF)

    return k(x5, pe, table)


def kernel(x, table):
    x5 = x.reshape(NW, NIBLK, IBLK, GCHUNK, GLEN)
    pe = _pos_encoding()
    return _embed_encode(x5, pe, table)
